# Initial kernel scaffold; baseline (speedup 1.0000x reference)
import jax
import jax.numpy as jnp
from jax.experimental import pallas as pl


def kernel(x, edge_index, W1, b1, g1, beta1, W2, b2, g2, beta2, W3, b3, g3, beta3):
    raise NotImplementedError("write your pallas kernel here")


# speedup vs baseline: 9.0818x; 9.0818x over previous
import functools

import jax
import jax.numpy as jnp
from jax import lax
from jax.experimental import pallas as pl
from jax.experimental.pallas import tpu as pltpu
from jax.experimental.pallas import tpu_sc as plsc

_NC = 2
_NS = 16
_NW = _NC * _NS
_CH = 128
_L = 16


def _sc_degree(dst, n_acc, steps):
    mesh = plsc.VectorSubcoreMesh(core_axis_name="c", subcore_axis_name="s")
    rpt = n_acc // _NS

    @functools.partial(
        pl.kernel,
        mesh=mesh,
        out_type=jax.ShapeDtypeStruct((_NC, n_acc, _L), jnp.float32),
        scratch_types=[
            pltpu.VMEM((_CH,), jnp.int32),
            pltpu.VMEM((_CH, _L), jnp.float32),
            pltpu.VMEM_SHARED((n_acc, _L), jnp.float32),
        ],
    )
    def k(dst_hbm, out_hbm, didx, ones, acc):
        c = lax.axis_index("c")
        s = lax.axis_index("s")
        wid = s * _NC + c

        def fill(i, val):
            def body(i, _):
                ones[i, :] = jnp.full((_L,), val, jnp.float32)
                return 0
            lax.fori_loop(0, i, body, 0)

        fill(_CH, 0.0)
        row0 = s * rpt
        nfull = rpt // _CH
        for t in range(nfull):
            pltpu.sync_copy(ones, acc.at[pl.ds(row0 + t * _CH, _CH)])
        rem = rpt - nfull * _CH
        if rem:
            pltpu.sync_copy(ones.at[pl.ds(0, rem)],
                            acc.at[pl.ds(row0 + nfull * _CH, rem)])
        fill(_CH, 1.0)
        plsc.subcore_barrier()

        base = wid * steps * _CH

        def body(i, _):
            off = base + i * _CH
            pltpu.sync_copy(dst_hbm.at[pl.ds(off, _CH)], didx)
            pltpu.sync_copy(ones, acc.at[didx], add=True)
            return 0

        lax.fori_loop(0, steps, body, 0)
        plsc.subcore_barrier()
        pltpu.sync_copy(acc.at[pl.ds(row0, rpt)],
                        out_hbm.at[c, pl.ds(row0, rpt)])

    return k(dst)


def _sc_scatter_rows(h, src, dst, n_acc, steps):
    d = h.shape[1]
    mesh = plsc.VectorSubcoreMesh(core_axis_name="c", subcore_axis_name="s")
    rpt = n_acc // _NS

    @functools.partial(
        pl.kernel,
        mesh=mesh,
        out_type=jax.ShapeDtypeStruct((_NC, n_acc, d), jnp.float32),
        scratch_types=[
            pltpu.VMEM((_CH,), jnp.int32),
            pltpu.VMEM((_CH,), jnp.int32),
            pltpu.VMEM((_CH, d), jnp.float32),
            pltpu.VMEM_SHARED((n_acc, d), jnp.float32),
            pltpu.SemaphoreType.DMA,
        ],
    )
    def k(h_hbm, src_hbm, dst_hbm, out_hbm, sidx, didx, rows, acc, sem):
        c = lax.axis_index("c")
        s = lax.axis_index("s")
        wid = s * _NC + c

        def zrow(i, _):
            for j in range(d // _L):
                rows[i, pl.ds(j * _L, _L)] = jnp.zeros((_L,), jnp.float32)
            return 0

        lax.fori_loop(0, _CH, zrow, 0)
        row0 = s * rpt
        nfull = rpt // _CH
        for t in range(nfull):
            pltpu.sync_copy(rows, acc.at[pl.ds(row0 + t * _CH, _CH)])
        rem = rpt - nfull * _CH
        if rem:
            pltpu.sync_copy(rows.at[pl.ds(0, rem)],
                            acc.at[pl.ds(row0 + nfull * _CH, rem)])
        plsc.subcore_barrier()

        base = wid * steps * _CH

        def body(i, _):
            off = base + i * _CH
            pltpu.sync_copy(src_hbm.at[pl.ds(off, _CH)], sidx)
            pltpu.sync_copy(dst_hbm.at[pl.ds(off, _CH)], didx)
            pltpu.async_copy(h_hbm.at[sidx], rows, sem).wait()
            pltpu.sync_copy(rows, acc.at[didx], add=True)
            return 0

        lax.fori_loop(0, steps, body, 0)
        plsc.subcore_barrier()
        pltpu.sync_copy(acc.at[pl.ds(row0, rpt)],
                        out_hbm.at[c, pl.ds(row0, rpt)])

    return k(h, src, dst)


def _dinv(deg_ref):
    degs = deg_ref[0] + deg_ref[1]
    return lax.rsqrt(degs[:, 0:1] + 1.0)


def _pre_body(x_ref, w_ref, deg_ref, o_ref):
    h = jnp.dot(x_ref[...], w_ref[...], preferred_element_type=jnp.float32)
    o_ref[...] = h * _dinv(deg_ref)


def _mid_body(agg_ref, hp_ref, deg_ref, b_ref, g_ref, be_ref, w_ref, o_ref):
    dinv = _dinv(deg_ref)
    a = (agg_ref[0] + agg_ref[1] + hp_ref[...]) * dinv + b_ref[...]
    mu = jnp.mean(a, axis=-1, keepdims=True)
    var = jnp.mean(jnp.square(a - mu), axis=-1, keepdims=True)
    z = (a - mu) * lax.rsqrt(var + 1e-5) * g_ref[...] + be_ref[...]
    z = jnp.maximum(z, 0.0)
    o_ref[...] = jnp.dot(z, w_ref[...],
                         preferred_element_type=jnp.float32) * dinv


def _fin_body(agg_ref, hp_ref, deg_ref, b_ref, g_ref, be_ref, o_ref):
    dinv = _dinv(deg_ref)
    a = (agg_ref[0] + agg_ref[1] + hp_ref[...]) * dinv + b_ref[...]
    mu = jnp.mean(a, axis=-1, keepdims=True)
    var = jnp.mean(jnp.square(a - mu), axis=-1, keepdims=True)
    o_ref[...] = (a - mu) * lax.rsqrt(var + 1e-5) * g_ref[...] + be_ref[...]


def _row_spec(r, d):
    return pl.BlockSpec((r, d), lambda i: (i, 0))


def _full_spec(shape):
    return pl.BlockSpec(shape, lambda i: tuple(0 for _ in shape))


def _acc_spec(r, d):
    return pl.BlockSpec((_NC, r, d), lambda i: (0, i, 0))


def _tc_pre(x, w, deg, r):
    n, d = x.shape
    return pl.pallas_call(
        _pre_body,
        grid=(n // r,),
        in_specs=[_row_spec(r, d), _full_spec((d, d)), _acc_spec(r, _L)],
        out_specs=_row_spec(r, d),
        out_shape=jax.ShapeDtypeStruct((n, d), jnp.float32),
    )(x, w, deg)


def _tc_mid(agg, hp, deg, b, g, be, w, r):
    n, d = hp.shape
    return pl.pallas_call(
        _mid_body,
        grid=(n // r,),
        in_specs=[_acc_spec(r, d), _row_spec(r, d), _acc_spec(r, _L),
                  _full_spec((1, d)), _full_spec((1, d)), _full_spec((1, d)),
                  _full_spec((d, d))],
        out_specs=_row_spec(r, d),
        out_shape=jax.ShapeDtypeStruct((n, d), jnp.float32),
    )(agg, hp, deg, b, g, be, w)


def _tc_fin(agg, hp, deg, b, g, be, r):
    n, d = hp.shape
    return pl.pallas_call(
        _fin_body,
        grid=(n // r,),
        in_specs=[_acc_spec(r, d), _row_spec(r, d), _acc_spec(r, _L),
                  _full_spec((1, d)), _full_spec((1, d)), _full_spec((1, d))],
        out_specs=_row_spec(r, d),
        out_shape=jax.ShapeDtypeStruct((n, d), jnp.float32),
    )(agg, hp, deg, b, g, be)


def kernel(x, edge_index, W1, b1, g1, beta1, W2, b2, g2, beta2,
           W3, b3, g3, beta3):
    n, d = x.shape
    e = edge_index.shape[1]

    steps = -(-e // (_NW * _CH))
    e_pad = steps * _NW * _CH
    n_acc = ((n + 1 + _NS * 8 - 1) // (_NS * 8)) * (_NS * 8)

    pad = e_pad - e
    src = jnp.concatenate([edge_index[0],
                           jnp.zeros((pad,), edge_index.dtype)])
    dst = jnp.concatenate([edge_index[1],
                           jnp.full((pad,), n, edge_index.dtype)])

    r = n
    for cand in (1024, 1000, 512, 500, 400, 256, 200, 128, 64, 8):
        if n % cand == 0 and cand % 8 == 0:
            r = cand
            break

    b1r, g1r, be1 = b1.reshape(1, d), g1.reshape(1, d), beta1.reshape(1, d)
    b2r, g2r, be2 = b2.reshape(1, d), g2.reshape(1, d), beta2.reshape(1, d)
    b3r, g3r, be3 = b3.reshape(1, d), g3.reshape(1, d), beta3.reshape(1, d)

    deg = _sc_degree(dst, n_acc, steps)

    h1 = _tc_pre(x, W1, deg, r)
    agg1 = _sc_scatter_rows(h1, src, dst, n_acc, steps)
    h2 = _tc_mid(agg1, h1, deg, b1r, g1r, be1, W2, r)
    agg2 = _sc_scatter_rows(h2, src, dst, n_acc, steps)
    h3 = _tc_mid(agg2, h2, deg, b2r, g2r, be2, W3, r)
    agg3 = _sc_scatter_rows(h3, src, dst, n_acc, steps)
    return _tc_fin(agg3, h3, deg, b3r, g3r, be3, r)
